# Initial kernel scaffold; baseline (speedup 1.0000x reference)
import jax
import jax.numpy as jnp
from jax.experimental import pallas as pl


def kernel(x, edge_index, etype, norm, bases, coeff, loop_weight, bias):
    raise NotImplementedError("write your pallas kernel here")


# speedup vs baseline: 11.3783x; 11.3783x over previous
import dataclasses
import functools

import jax
import jax.numpy as jnp
from jax import lax
from jax.experimental import pallas as pl
from jax.experimental.pallas import tpu as pltpu
from jax.experimental.pallas import tpu_sc as plsc

_N = 10000
_E = 320000
_D = 128
_R = 8
_B = 4

_NC = 2
_NS = 16
_NW = _NC * _NS
_CH = 128
_CPT = 79
_EPT = _CH * _CPT
_EPAD = _NW * _EPT
_NPAD = 10240
_RPT = _NPAD // _NS

_BN = 1000


def _xw_body(coeff_ref, bases_ref, x_ref, out_ref):
    w = coeff_ref[0, 0, 0] * bases_ref[0]
    for b in range(1, _B):
        w = w + coeff_ref[0, 0, b] * bases_ref[b]
    out_ref[0] = lax.dot_general(
        x_ref[...], w, (((1,), (0,)), ((), ())),
        preferred_element_type=jnp.float32)


_xw_call = pl.pallas_call(
    _xw_body,
    grid=(_R, _N // _BN),
    in_specs=[
        pl.BlockSpec((1, 1, _B), lambda r, i: (r, 0, 0)),
        pl.BlockSpec((_B, _D, _D), lambda r, i: (0, 0, 0)),
        pl.BlockSpec((_BN, _D), lambda r, i: (i, 0)),
    ],
    out_specs=pl.BlockSpec((1, _BN, _D), lambda r, i: (r, i, 0)),
    out_shape=jax.ShapeDtypeStruct((_R, _N, _D), jnp.float32),
)


_sc_mesh = plsc.VectorSubcoreMesh(core_axis_name="c", subcore_axis_name="s")

_sc_params = pltpu.CompilerParams()
if "needs_layout_passes" in pltpu.CompilerParams.__dataclass_fields__:
    _sc_params = dataclasses.replace(_sc_params, needs_layout_passes=False)


@functools.partial(
    pl.kernel,
    out_type=jax.ShapeDtypeStruct((_NC, _NPAD, _D), jnp.float32),
    mesh=_sc_mesh,
    compiler_params=_sc_params,
    scratch_types=[
        pltpu.VMEM((_CPT, _CH), jnp.int32),
        pltpu.VMEM((_CPT, _CH), jnp.int32),
        pltpu.VMEM((_CPT, _CH), jnp.float32),
        pltpu.VMEM((_CH, _D), jnp.float32),
        pltpu.VMEM_SHARED((_NPAD, _D), jnp.float32),
    ],
)
def _sc_scatter(table_hbm, gidx_hbm, dst_hbm, norm_hbm, zeros_hbm, out_hbm,
                gidx_v, dst_v, norm_v, rows_v, acc_sh):
    c = lax.axis_index("c")
    s = lax.axis_index("s")
    wid = c * _NS + s
    pltpu.sync_copy(gidx_hbm.at[wid], gidx_v)
    pltpu.sync_copy(dst_hbm.at[wid], dst_v)
    pltpu.sync_copy(norm_hbm.at[wid], norm_v)
    pltpu.sync_copy(zeros_hbm, acc_sh.at[pl.ds(s * _RPT, _RPT)])
    plsc.subcore_barrier()

    @pl.loop(0, _CPT)
    def _chunk(i):
        pltpu.sync_copy(table_hbm.at[gidx_v.at[i]], rows_v)

        @pl.loop(0, _CH)
        def _edge(e):
            nv = plsc.load_gather(
                norm_v, [jnp.full((16,), i, jnp.int32),
                         jnp.full((16,), e, jnp.int32)])
            for j in range(_D // 16):
                sl = (e, pl.ds(j * 16, 16))
                rows_v[sl] = rows_v[sl] * nv

        pltpu.sync_copy(rows_v, acc_sh.at[dst_v.at[i]], add=True)

    plsc.subcore_barrier()
    pltpu.sync_copy(acc_sh.at[pl.ds(s * _RPT, _RPT)],
                    out_hbm.at[c, pl.ds(s * _RPT, _RPT)])


def _fin_body(x_ref, lw_ref, bias_ref, agg_ref, out_ref):
    sl = lax.dot_general(
        x_ref[...], lw_ref[...], (((1,), (0,)), ((), ())),
        preferred_element_type=jnp.float32)
    out_ref[...] = jnp.maximum(
        sl + agg_ref[0] + agg_ref[1] + bias_ref[...], 0.0)


_fin_call = pl.pallas_call(
    _fin_body,
    grid=(_N // _BN,),
    in_specs=[
        pl.BlockSpec((_BN, _D), lambda i: (i, 0)),
        pl.BlockSpec((_D, _D), lambda i: (0, 0)),
        pl.BlockSpec((1, _D), lambda i: (0, 0)),
        pl.BlockSpec((_NC, _BN, _D), lambda i: (0, i, 0)),
    ],
    out_specs=pl.BlockSpec((_BN, _D), lambda i: (i, 0)),
    out_shape=jax.ShapeDtypeStruct((_N, _D), jnp.float32),
)


def kernel(x, edge_index, etype, norm, bases, coeff, loop_weight, bias):
    src = edge_index[0]
    dst = edge_index[1]
    gidx = etype.astype(jnp.int32) * _N + src.astype(jnp.int32)
    pad = _EPAD - _E
    gidx_p = jnp.concatenate(
        [gidx, jnp.zeros((pad,), jnp.int32)]).reshape(_NW, _CPT, _CH)
    dst_p = jnp.concatenate(
        [dst.astype(jnp.int32), jnp.zeros((pad,), jnp.int32)]
    ).reshape(_NW, _CPT, _CH)
    norm_p = jnp.concatenate(
        [norm[:, 0].astype(jnp.float32), jnp.zeros((pad,), jnp.float32)]
    ).reshape(_NW, _CPT, _CH)

    xw = _xw_call(coeff.reshape(_R, 1, _B), bases, x)
    table = xw.reshape(_R * _N, _D)
    zeros = jnp.zeros((_RPT, _D), jnp.float32)
    agg = _sc_scatter(table, gidx_p, dst_p, norm_p, zeros)
    return _fin_call(x, loop_weight, bias.reshape(1, _D), agg)
